# baseline (speedup 1.0000x reference)
import functools

import jax
import jax.numpy as jnp
from jax import lax
from jax.experimental import pallas as pl
from jax.experimental.pallas import tpu as pltpu
from jax.experimental.pallas import tpu_sc as plsc

NSP = 8192
NP = 65536
D = 512
NC = 2
NS = 16
NW = NC * NS
PPW = NP // NW
CHUNK = 16
NCH2 = PPW // (2 * CHUNK)
CH3 = 512
BLK2 = 512
F32 = jnp.float32
I32 = jnp.int32

_mesh = plsc.VectorSubcoreMesh(core_axis_name="c", subcore_axis_name="s",
                               num_cores=NC, num_subcores=NS)


def _precompute_body(s_ref, w1m_ref, w1a_ref, t1_ref, t2_ref):
    s = s_ref[...]
    t1_ref[:, :D] = s
    t1_ref[:, D:] = jnp.dot(s, w1m_ref[...], preferred_element_type=F32)
    t2_ref[:, :D] = s
    t2_ref[:, D:] = jnp.dot(s, w1a_ref[...], preferred_element_type=F32)


def _precompute(states, w1m, w1a):
    blk = 1024
    return pl.pallas_call(
        _precompute_body,
        grid=(NSP // blk,),
        in_specs=[
            pl.BlockSpec((blk, D), lambda i: (i, 0)),
            pl.BlockSpec((D, D), lambda i: (0, 0)),
            pl.BlockSpec((D, D), lambda i: (0, 0)),
        ],
        out_specs=[
            pl.BlockSpec((blk, 2 * D), lambda i: (i, 0)),
            pl.BlockSpec((blk, 2 * D), lambda i: (i, 0)),
        ],
        out_shape=[
            jax.ShapeDtypeStruct((NSP, 2 * D), F32),
            jax.ShapeDtypeStruct((NSP, 2 * D), F32),
        ],
        interpret=False,
    )(states, w1m, w1a)


def _phi_body(d_ref, s_ref, w1phi_ref, b1_ref, phi_ref):
    c = lax.broadcasted_iota(I32, (32, 1), 0)
    d_idx = c // 3
    s_idx = c - d_idx * 3
    oh_d = (d_idx == lax.broadcasted_iota(I32, (32, 10), 1)).astype(F32)
    oh_s = (s_idx == lax.broadcasted_iota(I32, (32, 3), 1)).astype(F32)
    emb = jnp.concatenate(
        [jnp.dot(oh_d, d_ref[...], preferred_element_type=F32),
         jnp.dot(oh_s, s_ref[...], preferred_element_type=F32)], axis=1)
    phi_ref[...] = (
        jnp.dot(emb, w1phi_ref[...], preferred_element_type=F32)
        + b1_ref[...][None, :])


def _phi_table(dist_table, speaker_table, w1phi, b1):
    return pl.pallas_call(
        _phi_body,
        out_shape=jax.ShapeDtypeStruct((32, D), F32),
        interpret=False,
    )(dist_table, speaker_table, w1phi, b1)


def _gather_body(ppw, t1_hbm, t2_hbm, scores_hbm, mid_hbm, aid_hbm,
                 pg_hbm, ssum_hbm,
                 mid_v, aid_v, scores_v, ssum_v,
                 bm0, ba0, bm1, ba1,
                 gsem0, gsem1, osem):
    wid = lax.axis_index("s") * NC + lax.axis_index("c")
    base = wid * ppw
    nch2 = ppw // (2 * CHUNK)
    pltpu.sync_copy(scores_hbm, scores_v)
    pltpu.sync_copy(mid_hbm.at[pl.ds(base, ppw)], mid_v)
    pltpu.sync_copy(aid_hbm.at[pl.ds(base, ppw)], aid_v)

    def sgrp(gi, carry):
        sl = pl.ds(gi * 16, 16)
        ssum_v[sl] = (plsc.load_gather(scores_v, [mid_v[sl]])
                      + plsc.load_gather(scores_v, [aid_v[sl]]))
        return carry

    lax.fori_loop(0, ppw // 16, sgrp, 0)
    pltpu.sync_copy(ssum_v, ssum_hbm.at[pl.ds(base, ppw)])

    def issue(ci, bm, ba, sem):
        lsl = pl.ds(ci * CHUNK, CHUNK)
        return [pltpu.async_copy(t1_hbm.at[mid_v[lsl]], bm, sem),
                pltpu.async_copy(t2_hbm.at[aid_v[lsl]], ba, sem)]

    def drain(sem, n):
        for _ in range(n):
            pltpu.make_async_copy(t1_hbm.at[pl.ds(0, CHUNK)], bm0,
                                  sem).wait()

    def vpass(bm, ba):
        def pair(j, carry):
            for k in range(D // 16):
                sl = pl.ds(k * 16, 16)
                sl2 = pl.ds(D + k * 16, 16)
                bm[j, sl] = bm[j, sl] * ba[j, sl]
                plsc.addupdate(bm.at[j, sl2], ba[j, sl2])
            return carry

        lax.fori_loop(0, CHUNK, pair, 0)

    issue(0, bm0, ba0, gsem0)

    def dchunk(t, carry):
        c0 = 2 * t
        off0 = base + c0 * CHUNK
        off1 = off0 + CHUNK
        drain(gsem0, 2)

        @pl.when(t > 0)
        def _():
            drain(osem, 1)

        d1 = issue(c0 + 1, bm1, ba1, gsem1)
        vpass(bm0, ba0)
        o1 = pltpu.async_copy(bm0, pg_hbm.at[pl.ds(off0, CHUNK)], osem)
        o1.wait()

        @pl.when(t < nch2 - 1)
        def _():
            issue(c0 + 2, bm0, ba0, gsem0)

        for d in d1:
            d.wait()
        vpass(bm1, ba1)
        pltpu.async_copy(bm1, pg_hbm.at[pl.ds(off1, CHUNK)], osem)
        return carry

    lax.fori_loop(0, nch2, dchunk, 0)
    drain(osem, 1)


def _gather(t1, t2, scores_flat, mid, aid):
    npairs = mid.shape[0]
    ppw = npairs // NW
    buf = lambda: pltpu.VMEM((CHUNK, 2 * D), F32)
    fn = pl.kernel(
        functools.partial(_gather_body, ppw),
        out_type=(
            jax.ShapeDtypeStruct((npairs, 2 * D), F32),
            jax.ShapeDtypeStruct((npairs,), F32),
        ),
        mesh=_mesh,
        compiler_params=pltpu.CompilerParams(needs_layout_passes=False),
        scratch_types=[
            pltpu.VMEM((ppw,), I32),
            pltpu.VMEM((ppw,), I32),
            pltpu.VMEM((NSP,), F32),
            pltpu.VMEM((ppw,), F32),
            buf(), buf(), buf(), buf(),
            pltpu.SemaphoreType.DMA,
            pltpu.SemaphoreType.DMA,
            pltpu.SemaphoreType.DMA,
        ],
    )
    return fn(t1, t2, scores_flat, mid, aid)


def _mlp_body(pg_ref, w1p_ref, phi_ref, w2_ref, ssum_ref, cmb_ref,
              b2_ref, coref_ref, expl_ref):
    ohT = (lax.broadcasted_iota(I32, (32, BLK2), 0)
           == cmb_ref[0, :, :]).astype(F32)
    pt = lax.dot_general(ohT, phi_ref[...],
                         dimension_numbers=(((0,), (0,)), ((), ())),
                         preferred_element_type=F32)
    h = jnp.maximum(
        jnp.dot(pg_ref[:, :D], w1p_ref[...], preferred_element_type=F32)
        + pg_ref[:, D:] + pt, 0.0)
    ps = jnp.sum(h * w2_ref[...][None, :], axis=1)
    cs = ps + ssum_ref[0, 0, :] + b2_ref[0]
    coref_ref[0, 0, :] = cs
    expl_ref[0, 0, :] = jnp.exp(cs)


def _mlp(pg, w1p, phi, w2_flat, ssum3, cmb3, b2):
    nblk = pg.shape[0] // BLK2
    return pl.pallas_call(
        _mlp_body,
        grid=(nblk,),
        in_specs=[
            pl.BlockSpec((BLK2, 2 * D), lambda i: (i, 0)),
            pl.BlockSpec((D, D), lambda i: (0, 0)),
            pl.BlockSpec((32, D), lambda i: (0, 0)),
            pl.BlockSpec((D,), lambda i: (0,)),
            pl.BlockSpec((1, 1, BLK2), lambda i: (i, 0, 0)),
            pl.BlockSpec((1, 1, BLK2), lambda i: (i, 0, 0)),
            pl.BlockSpec(memory_space=pltpu.SMEM),
        ],
        out_specs=[
            pl.BlockSpec((1, 1, BLK2), lambda i: (i, 0, 0)),
            pl.BlockSpec((1, 1, BLK2), lambda i: (i, 0, 0)),
        ],
        out_shape=[
            jax.ShapeDtypeStruct((nblk, 1, BLK2), F32),
            jax.ShapeDtypeStruct((nblk, 1, BLK2), F32),
        ],
        interpret=False,
    )(pg, w1p, phi, w2_flat, ssum3, cmb3, b2)


def _segsum_body(expl_hbm, mid_hbm, part_hbm, acc_v, mid_v, val_v):
    wid = lax.axis_index("s") * NC + lax.axis_index("c")
    base = wid * PPW

    def zero(i, carry):
        acc_v[pl.ds(i * 16, 16)] = jnp.zeros((16,), F32)
        return carry

    lax.fori_loop(0, NSP // 16, zero, 0)

    def chunk(ci, carry):
        off = base + ci * CH3
        pltpu.sync_copy(mid_hbm.at[pl.ds(off, CH3)], mid_v)
        pltpu.sync_copy(expl_hbm.at[pl.ds(off, CH3)], val_v)

        def grp(gi, carry2):
            sl = pl.ds(gi * 16, 16)
            plsc.addupdate_scatter(acc_v, [mid_v[sl]], val_v[sl])
            return carry2

        lax.fori_loop(0, CH3 // 16, grp, 0)
        return carry

    lax.fori_loop(0, PPW // CH3, chunk, 0)
    pltpu.sync_copy(acc_v, part_hbm.at[wid])


def _segsum(expl, mid):
    fn = pl.kernel(
        _segsum_body,
        out_type=jax.ShapeDtypeStruct((NW, NSP), F32),
        mesh=_mesh,
        compiler_params=pltpu.CompilerParams(needs_layout_passes=False),
        scratch_types=[
            pltpu.VMEM((NSP,), F32),
            pltpu.VMEM((CH3,), I32),
            pltpu.VMEM((CH3,), F32),
        ],
    )
    return fn(expl, mid)


def _denom_body(part_ref, r_ref):
    r_ref[...] = 1.0 / (jnp.sum(part_ref[...], axis=0) + 1.0)


def _denom(part):
    return pl.pallas_call(
        _denom_body,
        out_shape=jax.ShapeDtypeStruct((NSP,), F32),
        interpret=False,
    )(part)


def _probs_body(expl_hbm, mid_hbm, r_hbm, out_hbm, r_v, mid_v, val_v):
    wid = lax.axis_index("s") * NC + lax.axis_index("c")
    base = wid * PPW
    pltpu.sync_copy(r_hbm, r_v)

    def chunk(ci, carry):
        off = base + ci * CH3
        pltpu.sync_copy(mid_hbm.at[pl.ds(off, CH3)], mid_v)
        pltpu.sync_copy(expl_hbm.at[pl.ds(off, CH3)], val_v)

        def grp(gi, carry2):
            sl = pl.ds(gi * 16, 16)
            rg = plsc.load_gather(r_v, [mid_v[sl]])
            val_v[sl] = val_v[sl] * rg
            return carry2

        lax.fori_loop(0, CH3 // 16, grp, 0)
        pltpu.sync_copy(val_v, out_hbm.at[pl.ds(off, CH3)])
        return carry

    lax.fori_loop(0, PPW // CH3, chunk, 0)


def _probs(expl, mid, r):
    fn = pl.kernel(
        _probs_body,
        out_type=jax.ShapeDtypeStruct((NP,), F32),
        mesh=_mesh,
        compiler_params=pltpu.CompilerParams(needs_layout_passes=False),
        scratch_types=[
            pltpu.VMEM((NSP,), F32),
            pltpu.VMEM((CH3,), I32),
            pltpu.VMEM((CH3,), F32),
        ],
    )
    return fn(expl, mid, r)


def kernel(states_avg, scores, dist_table, speaker_table, W1, b1, W2, b2,
           mention_ids, antecedent_ids, distance_buckets, speakers):
    w1m = W1[0:D]
    w1a = W1[D:2 * D]
    w1p = W1[2 * D:3 * D]
    w1phi = W1[3 * D:]
    mid = mention_ids.astype(I32)
    aid = antecedent_ids.astype(I32)
    cmb = distance_buckets.astype(I32) * 3 + speakers.astype(I32)

    t1, t2 = _precompute(states_avg, w1m, w1a)
    phi = _phi_table(dist_table, speaker_table, w1phi, b1)
    pg, ssum = _gather(t1, t2, scores[:, 0], mid, aid)
    coref3, expl3 = _mlp(pg, w1p, phi, W2[:, 0],
                         ssum.reshape(NP // BLK2, 1, BLK2),
                         cmb.reshape(NP // BLK2, 1, BLK2), b2)
    expl = expl3.reshape(NP)
    part = _segsum(expl, mid)
    r = _denom(part)
    probs = _probs(expl, mid, r)
    return coref3.reshape(NP, 1), probs, r


# speedup vs baseline: 1.1316x; 1.1316x over previous
import functools

import jax
import jax.numpy as jnp
from jax import lax
from jax.experimental import pallas as pl
from jax.experimental.pallas import tpu as pltpu
from jax.experimental.pallas import tpu_sc as plsc

NSP = 8192
NP = 65536
D = 512
NC = 2
NS = 16
NW = NC * NS
PPW = NP // NW
CHUNK = 16
NCH2 = PPW // (2 * CHUNK)
CH3 = 512
BLK2 = 512
F32 = jnp.float32
I32 = jnp.int32

_mesh = plsc.VectorSubcoreMesh(core_axis_name="c", subcore_axis_name="s",
                               num_cores=NC, num_subcores=NS)


def _precompute_body(s_ref, w1m_ref, w1a_ref, t1_ref, t2_ref):
    s = s_ref[...]
    t1_ref[:, :D] = s
    t1_ref[:, D:] = jnp.dot(s, w1m_ref[...], preferred_element_type=F32)
    t2_ref[:, :D] = s
    t2_ref[:, D:] = jnp.dot(s, w1a_ref[...], preferred_element_type=F32)


def _precompute(states, w1m, w1a):
    blk = 1024
    return pl.pallas_call(
        _precompute_body,
        grid=(NSP // blk,),
        in_specs=[
            pl.BlockSpec((blk, D), lambda i: (i, 0)),
            pl.BlockSpec((D, D), lambda i: (0, 0)),
            pl.BlockSpec((D, D), lambda i: (0, 0)),
        ],
        out_specs=[
            pl.BlockSpec((blk, 2 * D), lambda i: (i, 0)),
            pl.BlockSpec((blk, 2 * D), lambda i: (i, 0)),
        ],
        out_shape=[
            jax.ShapeDtypeStruct((NSP, 2 * D), F32),
            jax.ShapeDtypeStruct((NSP, 2 * D), F32),
        ],
        interpret=False,
    )(states, w1m, w1a)


def _phi_body(d_ref, s_ref, w1phi_ref, b1_ref, phi_ref):
    c = lax.broadcasted_iota(I32, (32, 1), 0)
    d_idx = c // 3
    s_idx = c - d_idx * 3
    oh_d = (d_idx == lax.broadcasted_iota(I32, (32, 10), 1)).astype(F32)
    oh_s = (s_idx == lax.broadcasted_iota(I32, (32, 3), 1)).astype(F32)
    emb = jnp.concatenate(
        [jnp.dot(oh_d, d_ref[...], preferred_element_type=F32),
         jnp.dot(oh_s, s_ref[...], preferred_element_type=F32)], axis=1)
    phi_ref[...] = (
        jnp.dot(emb, w1phi_ref[...], preferred_element_type=F32)
        + b1_ref[...][None, :])


def _phi_table(dist_table, speaker_table, w1phi, b1):
    return pl.pallas_call(
        _phi_body,
        out_shape=jax.ShapeDtypeStruct((32, D), F32),
        interpret=False,
    )(dist_table, speaker_table, w1phi, b1)


def _gather_body(ppw, t1_hbm, t2_hbm, scores_hbm, mid_hbm, aid_hbm,
                 pg_hbm, ssum_hbm,
                 mid_v, aid_v, scores_v, ssum_v,
                 bm0, ba0, bm1, ba1,
                 gsem0, gsem1, osem):
    wid = lax.axis_index("s") * NC + lax.axis_index("c")
    base = wid * ppw
    nch2 = ppw // (2 * CHUNK)
    pltpu.sync_copy(scores_hbm, scores_v)
    pltpu.sync_copy(mid_hbm.at[pl.ds(base, ppw)], mid_v)
    pltpu.sync_copy(aid_hbm.at[pl.ds(base, ppw)], aid_v)

    def sgrp(gi, carry):
        sl = pl.ds(gi * 16, 16)
        ssum_v[sl] = (plsc.load_gather(scores_v, [mid_v[sl]])
                      + plsc.load_gather(scores_v, [aid_v[sl]]))
        return carry

    lax.fori_loop(0, ppw // 16, sgrp, 0)
    pltpu.sync_copy(ssum_v, ssum_hbm.at[pl.ds(base, ppw)])

    def issue(ci, bm, ba, sem):
        lsl = pl.ds(ci * CHUNK, CHUNK)
        return [pltpu.async_copy(t1_hbm.at[mid_v[lsl]], bm, sem),
                pltpu.async_copy(t2_hbm.at[aid_v[lsl]], ba, sem)]

    def drain(sem, n):
        for _ in range(n):
            pltpu.make_async_copy(t1_hbm.at[pl.ds(0, CHUNK)], bm0,
                                  sem).wait()

    def vpass(bm, ba):
        def pair(j, carry):
            for k in range(D // 16):
                sl = pl.ds(k * 16, 16)
                sl2 = pl.ds(D + k * 16, 16)
                bm[j, sl] = bm[j, sl] * ba[j, sl]
                bm[j, sl2] = bm[j, sl2] + ba[j, sl2]
            return carry

        lax.fori_loop(0, CHUNK, pair, 0)

    issue(0, bm0, ba0, gsem0)

    def dchunk(t, carry):
        c0 = 2 * t
        off0 = base + c0 * CHUNK
        off1 = off0 + CHUNK
        drain(gsem0, 2)

        @pl.when(t > 0)
        def _():
            drain(osem, 1)

        d1 = issue(c0 + 1, bm1, ba1, gsem1)
        vpass(bm0, ba0)
        o1 = pltpu.async_copy(bm0, pg_hbm.at[pl.ds(off0, CHUNK)], osem)
        o1.wait()

        @pl.when(t < nch2 - 1)
        def _():
            issue(c0 + 2, bm0, ba0, gsem0)

        for d in d1:
            d.wait()
        vpass(bm1, ba1)
        pltpu.async_copy(bm1, pg_hbm.at[pl.ds(off1, CHUNK)], osem)
        return carry

    lax.fori_loop(0, nch2, dchunk, 0)
    drain(osem, 1)


def _gather(t1, t2, scores_flat, mid, aid):
    npairs = mid.shape[0]
    ppw = npairs // NW
    buf = lambda: pltpu.VMEM((CHUNK, 2 * D), F32)
    fn = pl.kernel(
        functools.partial(_gather_body, ppw),
        out_type=(
            jax.ShapeDtypeStruct((npairs, 2 * D), F32),
            jax.ShapeDtypeStruct((npairs,), F32),
        ),
        mesh=_mesh,
        compiler_params=pltpu.CompilerParams(needs_layout_passes=False),
        scratch_types=[
            pltpu.VMEM((ppw,), I32),
            pltpu.VMEM((ppw,), I32),
            pltpu.VMEM((NSP,), F32),
            pltpu.VMEM((ppw,), F32),
            buf(), buf(), buf(), buf(),
            pltpu.SemaphoreType.DMA,
            pltpu.SemaphoreType.DMA,
            pltpu.SemaphoreType.DMA,
        ],
    )
    return fn(t1, t2, scores_flat, mid, aid)


def _mlp_body(pg_ref, w1p_ref, phi_ref, w2_ref, ssum_ref, cmb_ref,
              b2_ref, coref_ref, expl_ref):
    ohT = (lax.broadcasted_iota(I32, (32, BLK2), 0)
           == cmb_ref[0, :, :]).astype(F32)
    pt = lax.dot_general(ohT, phi_ref[...],
                         dimension_numbers=(((0,), (0,)), ((), ())),
                         preferred_element_type=F32)
    h = jnp.maximum(
        jnp.dot(pg_ref[:, :D], w1p_ref[...], preferred_element_type=F32)
        + pg_ref[:, D:] + pt, 0.0)
    ps = jnp.sum(h * w2_ref[...][None, :], axis=1)
    cs = ps + ssum_ref[0, 0, :] + b2_ref[0]
    coref_ref[0, 0, :] = cs
    expl_ref[0, 0, :] = jnp.exp(cs)


def _mlp(pg, w1p, phi, w2_flat, ssum3, cmb3, b2):
    nblk = pg.shape[0] // BLK2
    return pl.pallas_call(
        _mlp_body,
        grid=(nblk,),
        in_specs=[
            pl.BlockSpec((BLK2, 2 * D), lambda i: (i, 0)),
            pl.BlockSpec((D, D), lambda i: (0, 0)),
            pl.BlockSpec((32, D), lambda i: (0, 0)),
            pl.BlockSpec((D,), lambda i: (0,)),
            pl.BlockSpec((1, 1, BLK2), lambda i: (i, 0, 0)),
            pl.BlockSpec((1, 1, BLK2), lambda i: (i, 0, 0)),
            pl.BlockSpec(memory_space=pltpu.SMEM),
        ],
        out_specs=[
            pl.BlockSpec((1, 1, BLK2), lambda i: (i, 0, 0)),
            pl.BlockSpec((1, 1, BLK2), lambda i: (i, 0, 0)),
        ],
        out_shape=[
            jax.ShapeDtypeStruct((nblk, 1, BLK2), F32),
            jax.ShapeDtypeStruct((nblk, 1, BLK2), F32),
        ],
        interpret=False,
    )(pg, w1p, phi, w2_flat, ssum3, cmb3, b2)


def _segsum_body(expl_hbm, mid_hbm, part_hbm, acc_v, mid_v, val_v):
    wid = lax.axis_index("s") * NC + lax.axis_index("c")
    base = wid * PPW

    def zero(i, carry):
        acc_v[pl.ds(i * 16, 16)] = jnp.zeros((16,), F32)
        return carry

    lax.fori_loop(0, NSP // 16, zero, 0)

    def chunk(ci, carry):
        off = base + ci * CH3
        pltpu.sync_copy(mid_hbm.at[pl.ds(off, CH3)], mid_v)
        pltpu.sync_copy(expl_hbm.at[pl.ds(off, CH3)], val_v)

        def grp(gi, carry2):
            sl = pl.ds(gi * 16, 16)
            plsc.addupdate_scatter(acc_v, [mid_v[sl]], val_v[sl])
            return carry2

        lax.fori_loop(0, CH3 // 16, grp, 0)
        return carry

    lax.fori_loop(0, PPW // CH3, chunk, 0)
    pltpu.sync_copy(acc_v, part_hbm.at[wid])


def _segsum(expl, mid):
    fn = pl.kernel(
        _segsum_body,
        out_type=jax.ShapeDtypeStruct((NW, NSP), F32),
        mesh=_mesh,
        compiler_params=pltpu.CompilerParams(needs_layout_passes=False),
        scratch_types=[
            pltpu.VMEM((NSP,), F32),
            pltpu.VMEM((CH3,), I32),
            pltpu.VMEM((CH3,), F32),
        ],
    )
    return fn(expl, mid)


def _denom_body(part_ref, r_ref):
    r_ref[...] = 1.0 / (jnp.sum(part_ref[...], axis=0) + 1.0)


def _denom(part):
    return pl.pallas_call(
        _denom_body,
        out_shape=jax.ShapeDtypeStruct((NSP,), F32),
        interpret=False,
    )(part)


def _probs_body(expl_hbm, mid_hbm, r_hbm, out_hbm, r_v, mid_v, val_v):
    wid = lax.axis_index("s") * NC + lax.axis_index("c")
    base = wid * PPW
    pltpu.sync_copy(r_hbm, r_v)

    def chunk(ci, carry):
        off = base + ci * CH3
        pltpu.sync_copy(mid_hbm.at[pl.ds(off, CH3)], mid_v)
        pltpu.sync_copy(expl_hbm.at[pl.ds(off, CH3)], val_v)

        def grp(gi, carry2):
            sl = pl.ds(gi * 16, 16)
            rg = plsc.load_gather(r_v, [mid_v[sl]])
            val_v[sl] = val_v[sl] * rg
            return carry2

        lax.fori_loop(0, CH3 // 16, grp, 0)
        pltpu.sync_copy(val_v, out_hbm.at[pl.ds(off, CH3)])
        return carry

    lax.fori_loop(0, PPW // CH3, chunk, 0)


def _probs(expl, mid, r):
    fn = pl.kernel(
        _probs_body,
        out_type=jax.ShapeDtypeStruct((NP,), F32),
        mesh=_mesh,
        compiler_params=pltpu.CompilerParams(needs_layout_passes=False),
        scratch_types=[
            pltpu.VMEM((NSP,), F32),
            pltpu.VMEM((CH3,), I32),
            pltpu.VMEM((CH3,), F32),
        ],
    )
    return fn(expl, mid, r)


def kernel(states_avg, scores, dist_table, speaker_table, W1, b1, W2, b2,
           mention_ids, antecedent_ids, distance_buckets, speakers):
    w1m = W1[0:D]
    w1a = W1[D:2 * D]
    w1p = W1[2 * D:3 * D]
    w1phi = W1[3 * D:]
    mid = mention_ids.astype(I32)
    aid = antecedent_ids.astype(I32)
    cmb = distance_buckets.astype(I32) * 3 + speakers.astype(I32)

    t1, t2 = _precompute(states_avg, w1m, w1a)
    phi = _phi_table(dist_table, speaker_table, w1phi, b1)
    pg, ssum = _gather(t1, t2, scores[:, 0], mid, aid)
    coref3, expl3 = _mlp(pg, w1p, phi, W2[:, 0],
                         ssum.reshape(NP // BLK2, 1, BLK2),
                         cmb.reshape(NP // BLK2, 1, BLK2), b2)
    expl = expl3.reshape(NP)
    part = _segsum(expl, mid)
    r = _denom(part)
    probs = _probs(expl, mid, r)
    return coref3.reshape(NP, 1), probs, r


# speedup vs baseline: 1.6271x; 1.4378x over previous
import functools

import jax
import jax.numpy as jnp
from jax import lax
from jax.experimental import pallas as pl
from jax.experimental.pallas import tpu as pltpu
from jax.experimental.pallas import tpu_sc as plsc

NSP = 8192
NP = 65536
D = 512
NC = 2
NS = 16
NW = NC * NS
PPW = NP // NW
CHUNK = 16
NCH2 = PPW // (2 * CHUNK)
CH3 = 512
BLK2 = 512
F32 = jnp.float32
BF16 = jnp.bfloat16
I32 = jnp.int32

_mesh = plsc.VectorSubcoreMesh(core_axis_name="c", subcore_axis_name="s",
                               num_cores=NC, num_subcores=NS)


DW = D // 2


U32 = jnp.uint32


def _pack2(x):
    xb = x.astype(BF16).astype(F32)
    u = lax.bitcast_convert_type(xb, U32)
    lo = u[:, :DW] >> 16
    hi = u[:, DW:] & U32(0xFFFF0000)
    return lax.bitcast_convert_type(lo | hi, I32)


def _unpack2(w):
    u = lax.bitcast_convert_type(w, U32)
    lo = lax.bitcast_convert_type(u << 16, F32)
    hi = lax.bitcast_convert_type(u & U32(0xFFFF0000), F32)
    return jnp.concatenate([lo, hi], axis=1)


def _precompute_body(s_ref, w1m_ref, w1a_ref, t1_ref, t2_ref):
    s = s_ref[...]
    sp = _pack2(s)
    t1_ref[:, :DW] = sp
    t1_ref[:, DW:] = _pack2(jnp.dot(
        s, w1m_ref[...], preferred_element_type=F32))
    t2_ref[:, :DW] = sp
    t2_ref[:, DW:] = _pack2(jnp.dot(
        s, w1a_ref[...], preferred_element_type=F32))


def _precompute(states, w1m, w1a):
    blk = 1024
    return pl.pallas_call(
        _precompute_body,
        grid=(NSP // blk,),
        in_specs=[
            pl.BlockSpec((blk, D), lambda i: (i, 0)),
            pl.BlockSpec((D, D), lambda i: (0, 0)),
            pl.BlockSpec((D, D), lambda i: (0, 0)),
        ],
        out_specs=[
            pl.BlockSpec((blk, D), lambda i: (i, 0)),
            pl.BlockSpec((blk, D), lambda i: (i, 0)),
        ],
        out_shape=[
            jax.ShapeDtypeStruct((NSP, D), I32),
            jax.ShapeDtypeStruct((NSP, D), I32),
        ],
        interpret=False,
    )(states, w1m, w1a)


def _phi_body(d_ref, s_ref, w1phi_ref, b1_ref, phi_ref):
    c = lax.broadcasted_iota(I32, (32, 1), 0)
    d_idx = c // 3
    s_idx = c - d_idx * 3
    oh_d = (d_idx == lax.broadcasted_iota(I32, (32, 10), 1)).astype(F32)
    oh_s = (s_idx == lax.broadcasted_iota(I32, (32, 3), 1)).astype(F32)
    emb = jnp.concatenate(
        [jnp.dot(oh_d, d_ref[...], preferred_element_type=F32),
         jnp.dot(oh_s, s_ref[...], preferred_element_type=F32)], axis=1)
    phi_ref[...] = (
        jnp.dot(emb, w1phi_ref[...], preferred_element_type=F32)
        + b1_ref[...][None, :])


def _phi_table(dist_table, speaker_table, w1phi, b1):
    return pl.pallas_call(
        _phi_body,
        out_shape=jax.ShapeDtypeStruct((32, D), F32),
        interpret=False,
    )(dist_table, speaker_table, w1phi, b1)


def _gather_body(ppw, t1_hbm, t2_hbm, scores_hbm, mid_hbm, aid_hbm,
                 pg_hbm, ssum_hbm,
                 mid_v, aid_v, scores_v, ssum_v,
                 bm0, ba0, bm1, ba1,
                 gsem0, gsem1, osem):
    wid = lax.axis_index("s") * NC + lax.axis_index("c")
    base = wid * ppw
    nch2 = ppw // (2 * CHUNK)
    pltpu.sync_copy(scores_hbm, scores_v)
    pltpu.sync_copy(mid_hbm.at[pl.ds(base, ppw)], mid_v)
    pltpu.sync_copy(aid_hbm.at[pl.ds(base, ppw)], aid_v)

    def sgrp(gi, carry):
        sl = pl.ds(gi * 16, 16)
        ssum_v[sl] = (plsc.load_gather(scores_v, [mid_v[sl]])
                      + plsc.load_gather(scores_v, [aid_v[sl]]))
        return carry

    lax.fori_loop(0, ppw // 16, sgrp, 0)
    pltpu.sync_copy(ssum_v, ssum_hbm.at[pl.ds(base, ppw)])

    def issue(ci, bm, ba, sem):
        lsl = pl.ds(ci * CHUNK, CHUNK)
        return [pltpu.async_copy(t1_hbm.at[mid_v[lsl]], bm, sem),
                pltpu.async_copy(t2_hbm.at[aid_v[lsl]], ba, sem)]

    def drain(sem, n):
        for _ in range(n):
            pltpu.make_async_copy(t1_hbm.at[pl.ds(0, CHUNK)], bm0,
                                  sem).wait()

    def vpass(bm, ba):
        def pair(j, carry):
            for k in range(DW // 16):
                sl = pl.ds(k * 16, 16)
                sl2 = pl.ds(DW + k * 16, 16)
                m32 = plsc.bitcast(bm[j, sl], BF16)
                a32 = plsc.bitcast(ba[j, sl], BF16)
                bm[j, sl] = plsc.bitcast(m32 * a32, I32)
                g32 = (plsc.bitcast(bm[j, sl2], BF16)
                       + plsc.bitcast(ba[j, sl2], BF16))
                bm[j, sl2] = plsc.bitcast(g32, I32)
            return carry

        lax.fori_loop(0, CHUNK, pair, 0)

    issue(0, bm0, ba0, gsem0)

    def dchunk(t, carry):
        c0 = 2 * t
        off0 = base + c0 * CHUNK
        off1 = off0 + CHUNK
        drain(gsem0, 2)

        @pl.when(t > 0)
        def _():
            drain(osem, 1)

        d1 = issue(c0 + 1, bm1, ba1, gsem1)
        vpass(bm0, ba0)
        o1 = pltpu.async_copy(bm0, pg_hbm.at[pl.ds(off0, CHUNK)], osem)
        o1.wait()

        @pl.when(t < nch2 - 1)
        def _():
            issue(c0 + 2, bm0, ba0, gsem0)

        for d in d1:
            d.wait()
        vpass(bm1, ba1)
        pltpu.async_copy(bm1, pg_hbm.at[pl.ds(off1, CHUNK)], osem)
        return carry

    lax.fori_loop(0, nch2, dchunk, 0)
    drain(osem, 1)


def _gather(t1, t2, scores_flat, mid, aid):
    npairs = mid.shape[0]
    ppw = npairs // NW
    buf = lambda: pltpu.VMEM((CHUNK, D), I32)
    fn = pl.kernel(
        functools.partial(_gather_body, ppw),
        out_type=(
            jax.ShapeDtypeStruct((npairs, D), I32),
            jax.ShapeDtypeStruct((npairs,), F32),
        ),
        mesh=_mesh,
        compiler_params=pltpu.CompilerParams(needs_layout_passes=False),
        scratch_types=[
            pltpu.VMEM((ppw,), I32),
            pltpu.VMEM((ppw,), I32),
            pltpu.VMEM((NSP,), F32),
            pltpu.VMEM((ppw,), F32),
            buf(), buf(), buf(), buf(),
            pltpu.SemaphoreType.DMA,
            pltpu.SemaphoreType.DMA,
            pltpu.SemaphoreType.DMA,
        ],
    )
    return fn(t1, t2, scores_flat, mid, aid)


def _mlp_body(pg_ref, w1p_ref, phi_ref, w2_ref, ssum_ref, cmb_ref,
              b2_ref, coref_ref, expl_ref):
    ohT = (lax.broadcasted_iota(I32, (32, BLK2), 0)
           == cmb_ref[0, :, :]).astype(F32)
    pt = lax.dot_general(ohT, phi_ref[...],
                         dimension_numbers=(((0,), (0,)), ((), ())),
                         preferred_element_type=F32)
    prod_b = _unpack2(pg_ref[:, :DW]).astype(BF16)
    g_f = _unpack2(pg_ref[:, DW:])
    h = jnp.maximum(
        jnp.dot(prod_b, w1p_ref[...], preferred_element_type=F32)
        + g_f + pt, 0.0)
    ps = jnp.sum(h * w2_ref[...][None, :], axis=1)
    cs = ps + ssum_ref[0, 0, :] + b2_ref[0]
    coref_ref[0, 0, :] = cs
    expl_ref[0, 0, :] = jnp.exp(cs)


def _mlp(pg, w1p, phi, w2_flat, ssum3, cmb3, b2):
    nblk = pg.shape[0] // BLK2
    return pl.pallas_call(
        _mlp_body,
        grid=(nblk,),
        in_specs=[
            pl.BlockSpec((BLK2, D), lambda i: (i, 0)),
            pl.BlockSpec((D, D), lambda i: (0, 0)),
            pl.BlockSpec((32, D), lambda i: (0, 0)),
            pl.BlockSpec((D,), lambda i: (0,)),
            pl.BlockSpec((1, 1, BLK2), lambda i: (i, 0, 0)),
            pl.BlockSpec((1, 1, BLK2), lambda i: (i, 0, 0)),
            pl.BlockSpec(memory_space=pltpu.SMEM),
        ],
        out_specs=[
            pl.BlockSpec((1, 1, BLK2), lambda i: (i, 0, 0)),
            pl.BlockSpec((1, 1, BLK2), lambda i: (i, 0, 0)),
        ],
        out_shape=[
            jax.ShapeDtypeStruct((nblk, 1, BLK2), F32),
            jax.ShapeDtypeStruct((nblk, 1, BLK2), F32),
        ],
        interpret=False,
    )(pg, w1p, phi, w2_flat, ssum3, cmb3, b2)


def _segsum_body(expl_hbm, mid_hbm, part_hbm, acc_v, mid_v, val_v):
    wid = lax.axis_index("s") * NC + lax.axis_index("c")
    base = wid * PPW

    def zero(i, carry):
        acc_v[pl.ds(i * 16, 16)] = jnp.zeros((16,), F32)
        return carry

    lax.fori_loop(0, NSP // 16, zero, 0)

    def chunk(ci, carry):
        off = base + ci * CH3
        pltpu.sync_copy(mid_hbm.at[pl.ds(off, CH3)], mid_v)
        pltpu.sync_copy(expl_hbm.at[pl.ds(off, CH3)], val_v)

        def grp(gi, carry2):
            sl = pl.ds(gi * 16, 16)
            plsc.addupdate_scatter(acc_v, [mid_v[sl]], val_v[sl])
            return carry2

        lax.fori_loop(0, CH3 // 16, grp, 0)
        return carry

    lax.fori_loop(0, PPW // CH3, chunk, 0)
    pltpu.sync_copy(acc_v, part_hbm.at[wid])


def _segsum(expl, mid):
    fn = pl.kernel(
        _segsum_body,
        out_type=jax.ShapeDtypeStruct((NW, NSP), F32),
        mesh=_mesh,
        compiler_params=pltpu.CompilerParams(needs_layout_passes=False),
        scratch_types=[
            pltpu.VMEM((NSP,), F32),
            pltpu.VMEM((CH3,), I32),
            pltpu.VMEM((CH3,), F32),
        ],
    )
    return fn(expl, mid)


def _denom_body(part_ref, r_ref):
    r_ref[...] = 1.0 / (jnp.sum(part_ref[...], axis=0) + 1.0)


def _denom(part):
    return pl.pallas_call(
        _denom_body,
        out_shape=jax.ShapeDtypeStruct((NSP,), F32),
        interpret=False,
    )(part)


def _probs_body(expl_hbm, mid_hbm, r_hbm, out_hbm, r_v, mid_v, val_v):
    wid = lax.axis_index("s") * NC + lax.axis_index("c")
    base = wid * PPW
    pltpu.sync_copy(r_hbm, r_v)

    def chunk(ci, carry):
        off = base + ci * CH3
        pltpu.sync_copy(mid_hbm.at[pl.ds(off, CH3)], mid_v)
        pltpu.sync_copy(expl_hbm.at[pl.ds(off, CH3)], val_v)

        def grp(gi, carry2):
            sl = pl.ds(gi * 16, 16)
            rg = plsc.load_gather(r_v, [mid_v[sl]])
            val_v[sl] = val_v[sl] * rg
            return carry2

        lax.fori_loop(0, CH3 // 16, grp, 0)
        pltpu.sync_copy(val_v, out_hbm.at[pl.ds(off, CH3)])
        return carry

    lax.fori_loop(0, PPW // CH3, chunk, 0)


def _probs(expl, mid, r):
    fn = pl.kernel(
        _probs_body,
        out_type=jax.ShapeDtypeStruct((NP,), F32),
        mesh=_mesh,
        compiler_params=pltpu.CompilerParams(needs_layout_passes=False),
        scratch_types=[
            pltpu.VMEM((NSP,), F32),
            pltpu.VMEM((CH3,), I32),
            pltpu.VMEM((CH3,), F32),
        ],
    )
    return fn(expl, mid, r)


def kernel(states_avg, scores, dist_table, speaker_table, W1, b1, W2, b2,
           mention_ids, antecedent_ids, distance_buckets, speakers):
    w1m = W1[0:D]
    w1a = W1[D:2 * D]
    w1p = W1[2 * D:3 * D].astype(BF16)
    w1phi = W1[3 * D:]
    mid = mention_ids.astype(I32)
    aid = antecedent_ids.astype(I32)
    cmb = distance_buckets.astype(I32) * 3 + speakers.astype(I32)

    t1, t2 = _precompute(states_avg, w1m, w1a)
    phi = _phi_table(dist_table, speaker_table, w1phi, b1)
    pg, ssum = _gather(t1, t2, scores[:, 0], mid, aid)
    coref3, expl3 = _mlp(pg, w1p, phi, W2[:, 0],
                         ssum.reshape(NP // BLK2, 1, BLK2),
                         cmb.reshape(NP // BLK2, 1, BLK2), b2)
    expl = expl3.reshape(NP)
    part = _segsum(expl, mid)
    r = _denom(part)
    probs = _probs(expl, mid, r)
    return coref3.reshape(NP, 1), probs, r


# speedup vs baseline: 1.7482x; 1.0744x over previous
import functools

import jax
import jax.numpy as jnp
from jax import lax
from jax.experimental import pallas as pl
from jax.experimental.pallas import tpu as pltpu
from jax.experimental.pallas import tpu_sc as plsc

NSP = 8192
NP = 65536
D = 512
NC = 2
NS = 16
NW = NC * NS
PPW = NP // NW
CHUNK = 32
NCH2 = PPW // (2 * CHUNK)
CH3 = 512
BLK2 = 512
F32 = jnp.float32
BF16 = jnp.bfloat16
I32 = jnp.int32

_mesh = plsc.VectorSubcoreMesh(core_axis_name="c", subcore_axis_name="s",
                               num_cores=NC, num_subcores=NS)


DW = D // 2


U32 = jnp.uint32


def _pack2(x):
    xb = x.astype(BF16).astype(F32)
    u = lax.bitcast_convert_type(xb, U32)
    lo = u[:, :DW] >> 16
    hi = u[:, DW:] & U32(0xFFFF0000)
    return lax.bitcast_convert_type(lo | hi, I32)


def _unpack2(w):
    u = lax.bitcast_convert_type(w, U32)
    lo = lax.bitcast_convert_type(u << 16, F32)
    hi = lax.bitcast_convert_type(u & U32(0xFFFF0000), F32)
    return jnp.concatenate([lo, hi], axis=1)


def _precompute_body(s_ref, w1m_ref, w1a_ref, d_ref, k_ref, w1phi_ref,
                     b1_ref, t1_ref, t2_ref, phi_ref):
    s = s_ref[...]
    sp = _pack2(s)
    t1_ref[:, :DW] = sp
    t1_ref[:, DW:] = _pack2(jnp.dot(
        s, w1m_ref[...], preferred_element_type=F32))
    t2_ref[:, :DW] = sp
    t2_ref[:, DW:] = _pack2(jnp.dot(
        s, w1a_ref[...], preferred_element_type=F32))

    @pl.when(pl.program_id(0) == 0)
    def _():
        c = lax.broadcasted_iota(I32, (32, 1), 0)
        d_idx = c // 3
        s_idx = c - d_idx * 3
        oh_d = (d_idx == lax.broadcasted_iota(I32, (32, 10), 1)).astype(F32)
        oh_s = (s_idx == lax.broadcasted_iota(I32, (32, 3), 1)).astype(F32)
        emb = jnp.concatenate(
            [jnp.dot(oh_d, d_ref[...], preferred_element_type=F32),
             jnp.dot(oh_s, k_ref[...], preferred_element_type=F32)], axis=1)
        phi_ref[...] = (
            jnp.dot(emb, w1phi_ref[...], preferred_element_type=F32)
            + b1_ref[...][None, :])


def _precompute(states, w1m, w1a, dist_table, speaker_table, w1phi, b1):
    blk = 1024
    return pl.pallas_call(
        _precompute_body,
        grid=(NSP // blk,),
        in_specs=[
            pl.BlockSpec((blk, D), lambda i: (i, 0)),
            pl.BlockSpec((D, D), lambda i: (0, 0)),
            pl.BlockSpec((D, D), lambda i: (0, 0)),
            pl.BlockSpec((10, 20), lambda i: (0, 0)),
            pl.BlockSpec((3, 20), lambda i: (0, 0)),
            pl.BlockSpec((40, D), lambda i: (0, 0)),
            pl.BlockSpec((D,), lambda i: (0,)),
        ],
        out_specs=[
            pl.BlockSpec((blk, D), lambda i: (i, 0)),
            pl.BlockSpec((blk, D), lambda i: (i, 0)),
            pl.BlockSpec((32, D), lambda i: (0, 0)),
        ],
        out_shape=[
            jax.ShapeDtypeStruct((NSP, D), I32),
            jax.ShapeDtypeStruct((NSP, D), I32),
            jax.ShapeDtypeStruct((32, D), F32),
        ],
        interpret=False,
    )(states, w1m, w1a, dist_table, speaker_table, w1phi, b1)


def _gather_body(ppw, t1_hbm, t2_hbm, scores_hbm, mid_hbm, aid_hbm,
                 pg_hbm, ssum_hbm,
                 mid_v, aid_v, scores_v, ssum_v,
                 bm0, ba0, bm1, ba1,
                 gsem0, gsem1, osem):
    wid = lax.axis_index("s") * NC + lax.axis_index("c")
    base = wid * ppw
    nch2 = ppw // (2 * CHUNK)
    pltpu.sync_copy(scores_hbm, scores_v)
    pltpu.sync_copy(mid_hbm.at[pl.ds(base, ppw)], mid_v)
    pltpu.sync_copy(aid_hbm.at[pl.ds(base, ppw)], aid_v)

    def sgrp(gi, carry):
        sl = pl.ds(gi * 16, 16)
        ssum_v[sl] = (plsc.load_gather(scores_v, [mid_v[sl]])
                      + plsc.load_gather(scores_v, [aid_v[sl]]))
        return carry

    lax.fori_loop(0, ppw // 16, sgrp, 0)
    pltpu.sync_copy(ssum_v, ssum_hbm.at[pl.ds(base, ppw)])

    def issue(ci, bm, ba, sem):
        ds = []
        for u in range(CHUNK // 16):
            lsl = pl.ds(ci * CHUNK + u * 16, 16)
            osl = pl.ds(u * 16, 16)
            ds.append(pltpu.async_copy(t1_hbm.at[mid_v[lsl]],
                                       bm.at[osl], sem))
            ds.append(pltpu.async_copy(t2_hbm.at[aid_v[lsl]],
                                       ba.at[osl], sem))
        return ds

    def drain(sem, n, rows):
        for _ in range(n):
            pltpu.make_async_copy(t1_hbm.at[pl.ds(0, rows)],
                                  bm0.at[pl.ds(0, rows)], sem).wait()

    def vpass(bm, ba):
        def pair(j, carry):
            for k in range(DW // 16):
                sl = pl.ds(k * 16, 16)
                sl2 = pl.ds(DW + k * 16, 16)
                m32 = plsc.bitcast(bm[j, sl], BF16)
                a32 = plsc.bitcast(ba[j, sl], BF16)
                bm[j, sl] = plsc.bitcast(m32 * a32, I32)
                g32 = (plsc.bitcast(bm[j, sl2], BF16)
                       + plsc.bitcast(ba[j, sl2], BF16))
                bm[j, sl2] = plsc.bitcast(g32, I32)
            return carry

        lax.fori_loop(0, CHUNK, pair, 0)

    issue(0, bm0, ba0, gsem0)

    def dchunk(t, carry):
        c0 = 2 * t
        off0 = base + c0 * CHUNK
        off1 = off0 + CHUNK
        drain(gsem0, CHUNK // 8, 16)

        @pl.when(t > 0)
        def _():
            drain(osem, 1, CHUNK)

        d1 = issue(c0 + 1, bm1, ba1, gsem1)
        vpass(bm0, ba0)
        o1 = pltpu.async_copy(bm0, pg_hbm.at[pl.ds(off0, CHUNK)], osem)
        o1.wait()

        @pl.when(t < nch2 - 1)
        def _():
            issue(c0 + 2, bm0, ba0, gsem0)

        for d in d1:
            d.wait()
        vpass(bm1, ba1)
        pltpu.async_copy(bm1, pg_hbm.at[pl.ds(off1, CHUNK)], osem)
        return carry

    lax.fori_loop(0, nch2, dchunk, 0)
    drain(osem, 1, CHUNK)


def _gather(t1, t2, scores_flat, mid, aid):
    npairs = mid.shape[0]
    ppw = npairs // NW
    buf = lambda: pltpu.VMEM((CHUNK, D), I32)
    fn = pl.kernel(
        functools.partial(_gather_body, ppw),
        out_type=(
            jax.ShapeDtypeStruct((npairs, D), I32),
            jax.ShapeDtypeStruct((npairs,), F32),
        ),
        mesh=_mesh,
        compiler_params=pltpu.CompilerParams(needs_layout_passes=False),
        scratch_types=[
            pltpu.VMEM((ppw,), I32),
            pltpu.VMEM((ppw,), I32),
            pltpu.VMEM((NSP,), F32),
            pltpu.VMEM((ppw,), F32),
            buf(), buf(), buf(), buf(),
            pltpu.SemaphoreType.DMA,
            pltpu.SemaphoreType.DMA,
            pltpu.SemaphoreType.DMA,
        ],
    )
    return fn(t1, t2, scores_flat, mid, aid)


def _mlp_body(pg_ref, w1p_ref, phi_ref, w2_ref, ssum_ref, cmb_ref,
              b2_ref, coref_ref, expl_ref):
    ohT = (lax.broadcasted_iota(I32, (32, BLK2), 0)
           == cmb_ref[0, :, :]).astype(F32)
    pt = lax.dot_general(ohT, phi_ref[...],
                         dimension_numbers=(((0,), (0,)), ((), ())),
                         preferred_element_type=F32)
    prod_b = _unpack2(pg_ref[:, :DW]).astype(BF16)
    g_f = _unpack2(pg_ref[:, DW:])
    h = jnp.maximum(
        jnp.dot(prod_b, w1p_ref[...], preferred_element_type=F32)
        + g_f + pt, 0.0)
    ps = jnp.sum(h * w2_ref[...][None, :], axis=1)
    cs = ps + ssum_ref[0, 0, :] + b2_ref[0]
    coref_ref[0, 0, :] = cs
    expl_ref[0, 0, :] = jnp.exp(cs)


def _mlp(pg, w1p, phi, w2_flat, ssum3, cmb3, b2):
    nblk = pg.shape[0] // BLK2
    return pl.pallas_call(
        _mlp_body,
        grid=(nblk,),
        in_specs=[
            pl.BlockSpec((BLK2, D), lambda i: (i, 0)),
            pl.BlockSpec((D, D), lambda i: (0, 0)),
            pl.BlockSpec((32, D), lambda i: (0, 0)),
            pl.BlockSpec((D,), lambda i: (0,)),
            pl.BlockSpec((1, 1, BLK2), lambda i: (i, 0, 0)),
            pl.BlockSpec((1, 1, BLK2), lambda i: (i, 0, 0)),
            pl.BlockSpec(memory_space=pltpu.SMEM),
        ],
        out_specs=[
            pl.BlockSpec((1, 1, BLK2), lambda i: (i, 0, 0)),
            pl.BlockSpec((1, 1, BLK2), lambda i: (i, 0, 0)),
        ],
        out_shape=[
            jax.ShapeDtypeStruct((nblk, 1, BLK2), F32),
            jax.ShapeDtypeStruct((nblk, 1, BLK2), F32),
        ],
        interpret=False,
    )(pg, w1p, phi, w2_flat, ssum3, cmb3, b2)


def _segsum_body(expl_hbm, mid_hbm, part_hbm, acc_v, mid_v, val_v):
    wid = lax.axis_index("s") * NC + lax.axis_index("c")
    base = wid * PPW

    def zero(i, carry):
        acc_v[pl.ds(i * 16, 16)] = jnp.zeros((16,), F32)
        return carry

    lax.fori_loop(0, NSP // 16, zero, 0)

    def chunk(ci, carry):
        off = base + ci * CH3
        pltpu.sync_copy(mid_hbm.at[pl.ds(off, CH3)], mid_v)
        pltpu.sync_copy(expl_hbm.at[pl.ds(off, CH3)], val_v)

        def grp(gi, carry2):
            sl = pl.ds(gi * 16, 16)
            plsc.addupdate_scatter(acc_v, [mid_v[sl]], val_v[sl])
            return carry2

        lax.fori_loop(0, CH3 // 16, grp, 0)
        return carry

    lax.fori_loop(0, PPW // CH3, chunk, 0)
    pltpu.sync_copy(acc_v, part_hbm.at[wid])


def _segsum(expl, mid):
    fn = pl.kernel(
        _segsum_body,
        out_type=jax.ShapeDtypeStruct((NW, NSP), F32),
        mesh=_mesh,
        compiler_params=pltpu.CompilerParams(needs_layout_passes=False),
        scratch_types=[
            pltpu.VMEM((NSP,), F32),
            pltpu.VMEM((CH3,), I32),
            pltpu.VMEM((CH3,), F32),
        ],
    )
    return fn(expl, mid)


def _denom_body(part_ref, r_ref):
    r_ref[...] = 1.0 / (jnp.sum(part_ref[...], axis=0) + 1.0)


def _denom(part):
    return pl.pallas_call(
        _denom_body,
        out_shape=jax.ShapeDtypeStruct((NSP,), F32),
        interpret=False,
    )(part)


def _probs_body(expl_hbm, mid_hbm, r_hbm, out_hbm, r_v, mid_v, val_v):
    wid = lax.axis_index("s") * NC + lax.axis_index("c")
    base = wid * PPW
    pltpu.sync_copy(r_hbm, r_v)

    def chunk(ci, carry):
        off = base + ci * CH3
        pltpu.sync_copy(mid_hbm.at[pl.ds(off, CH3)], mid_v)
        pltpu.sync_copy(expl_hbm.at[pl.ds(off, CH3)], val_v)

        def grp(gi, carry2):
            sl = pl.ds(gi * 16, 16)
            rg = plsc.load_gather(r_v, [mid_v[sl]])
            val_v[sl] = val_v[sl] * rg
            return carry2

        lax.fori_loop(0, CH3 // 16, grp, 0)
        pltpu.sync_copy(val_v, out_hbm.at[pl.ds(off, CH3)])
        return carry

    lax.fori_loop(0, PPW // CH3, chunk, 0)


def _probs(expl, mid, r):
    fn = pl.kernel(
        _probs_body,
        out_type=jax.ShapeDtypeStruct((NP,), F32),
        mesh=_mesh,
        compiler_params=pltpu.CompilerParams(needs_layout_passes=False),
        scratch_types=[
            pltpu.VMEM((NSP,), F32),
            pltpu.VMEM((CH3,), I32),
            pltpu.VMEM((CH3,), F32),
        ],
    )
    return fn(expl, mid, r)


def kernel(states_avg, scores, dist_table, speaker_table, W1, b1, W2, b2,
           mention_ids, antecedent_ids, distance_buckets, speakers):
    w1m = W1[0:D]
    w1a = W1[D:2 * D]
    w1p = W1[2 * D:3 * D].astype(BF16)
    w1phi = W1[3 * D:]
    mid = mention_ids.astype(I32)
    aid = antecedent_ids.astype(I32)
    cmb = distance_buckets.astype(I32) * 3 + speakers.astype(I32)

    t1, t2, phi = _precompute(states_avg, w1m, w1a, dist_table,
                              speaker_table, w1phi, b1)
    pg, ssum = _gather(t1, t2, scores[:, 0], mid, aid)
    coref3, expl3 = _mlp(pg, w1p, phi, W2[:, 0],
                         ssum.reshape(NP // BLK2, 1, BLK2),
                         cmb.reshape(NP // BLK2, 1, BLK2), b2)
    expl = expl3.reshape(NP)
    part = _segsum(expl, mid)
    r = _denom(part)
    probs = _probs(expl, mid, r)
    return coref3.reshape(NP, 1), probs, r
